# baseline (speedup 1.0000x reference)
import jax
import jax.numpy as jnp
from jax.experimental import pallas as pl
from jax.experimental.pallas import tpu as pltpu

N = 10000
BM = 200
R = N // BM
G = 5
RG = R // G


def _dot(a, b):
    return jax.lax.dot(a, b, precision=jax.lax.Precision.DEFAULT,
                       preferred_element_type=jnp.float32)


def _bdot(a, b):
    return jax.lax.dot(a.astype(jnp.bfloat16), b,
                       preferred_element_type=jnp.float32)


def _gcn_kernel(x_ref, adj_ref, adjc_ref, w1_ref, w2_ref, b1_ref, b2_ref,
                lg_ref, zg_ref, lc_ref, zc_ref, hg_ref, hc_ref,
                hstage_ref, s1s2g_ref, s2c_ref):
    p = pl.program_id(0)
    i = pl.program_id(1)
    rows = pl.ds(i * BM, BM)
    sub = pl.ds((i % G) * BM, BM)

    @pl.when((p == 0) & (i == 0))
    def _compute_s1():
        s1s2g_ref[...] = _dot(x_ref[...], w1_ref[...]).astype(jnp.bfloat16)

    @pl.when(p == 0)
    def _layer1():
        s1 = s1s2g_ref[...]
        b1 = b1_ref[...]
        hg = jax.nn.relu(_bdot(adj_ref[...], s1) + b1)
        hc = jax.nn.relu(_bdot(adjc_ref[...], s1) + b1)
        hg_ref[sub, :] = hg
        hc_ref[sub, :] = hc
        hstage_ref[rows, 0:16] = hg
        hstage_ref[rows, 16:32] = hc

    @pl.when((p == 1) & (i == 0))
    def _compute_s2():
        w2 = w2_ref[...]
        s2g_ref = s1s2g_ref
        s2g_ref[...] = _dot(hstage_ref[:, 0:16], w2).astype(jnp.bfloat16)
        s2c_ref[...] = _dot(hstage_ref[:, 16:32], w2).astype(jnp.bfloat16)

    @pl.when(p == 1)
    def _layer2():
        b2 = b2_ref[...]
        ones = jnp.ones((16, 16), jnp.float32)
        zg = _bdot(adj_ref[...], s1s2g_ref[...]) + b2
        zc = _bdot(adjc_ref[...], s2c_ref[...]) + b2
        sg = _dot(jnp.exp(zg), ones)
        sc = _dot(jnp.exp(zc), ones)
        zg_ref[sub, :] = zg
        lg_ref[sub, :] = zg - jnp.log(sg)
        zc_ref[sub, :] = zc
        lc_ref[sub, :] = zc - jnp.log(sc)


def kernel(x, adj, adj_CNN, W1, b1, W2, b2):
    nfeat = x.shape[1]
    nhid = W1.shape[1]
    ncls = W2.shape[1]
    b1r = b1.reshape(1, nhid)
    b2r = b2.reshape(1, ncls)

    grid = (2, R)
    blk_adj = pl.BlockSpec((BM, N), lambda p, i: (i, 0))
    const = lambda r, c: pl.BlockSpec((r, c), lambda p, i: (0, 0))
    blk_p0 = lambda c: pl.BlockSpec(
        (G * BM, c), lambda p, i: ((1 - p) * (i // G) + p * (RG - 1), 0))
    blk_p1 = lambda c: pl.BlockSpec((G * BM, c), lambda p, i: (p * (i // G), 0))

    out = pl.pallas_call(
        _gcn_kernel,
        grid=grid,
        in_specs=[const(N, nfeat), blk_adj, blk_adj,
                  const(nfeat, nhid), const(nhid, ncls),
                  const(1, nhid), const(1, ncls)],
        out_specs=[blk_p1(ncls), blk_p1(ncls), blk_p1(ncls), blk_p1(ncls),
                   blk_p0(nhid), blk_p0(nhid)],
        out_shape=[jax.ShapeDtypeStruct((N, ncls), jnp.float32)] * 4
                  + [jax.ShapeDtypeStruct((N, nhid), jnp.float32)] * 2,
        scratch_shapes=[
            pltpu.VMEM((N, 128), jnp.float32),
            pltpu.VMEM((N, nhid), jnp.bfloat16),
            pltpu.VMEM((N, ncls), jnp.bfloat16),
        ],
        compiler_params=pltpu.CompilerParams(
            dimension_semantics=("arbitrary", "arbitrary"),
        ),
    )(x, adj, adj_CNN, W1, W2, b1r, b2r)

    lsm_g, z_g, lsm_c, z_c, h_g, h_c = out
    return (lsm_g, z_g, lsm_c, z_c, h_g, h_c)


# speedup vs baseline: 1.0045x; 1.0045x over previous
import jax
import jax.numpy as jnp
from jax.experimental import pallas as pl
from jax.experimental.pallas import tpu as pltpu

N = 10000
BM = 200
R = N // BM
G = 5
RG = R // G


def _dot(a, b):
    return jax.lax.dot(a, b, precision=jax.lax.Precision.DEFAULT,
                       preferred_element_type=jnp.float32)


def _gcn_kernel(x_ref, adj_ref, adjc_ref, w1_ref, w2_ref, b1_ref, b2_ref,
                lg_ref, zg_ref, lc_ref, zc_ref, hg_ref, hc_ref,
                s1_ref, s2g_ref, s2c_ref):
    p = pl.program_id(0)
    i = pl.program_id(1)
    rows = pl.ds(i * BM, BM)
    sub = pl.ds((i % G) * BM, BM)

    @pl.when((p == 0) & (i == 0))
    def _compute_s1():
        s1_ref[...] = _dot(x_ref[...], w1_ref[...])

    @pl.when(p == 0)
    def _layer1():
        s1 = s1_ref[...]
        b1 = b1_ref[...]
        hg = jax.nn.relu(_dot(adj_ref[...], s1) + b1)
        hc = jax.nn.relu(_dot(adjc_ref[...], s1) + b1)
        hg_ref[sub, :] = hg
        hc_ref[sub, :] = hc
        w2 = w2_ref[...]
        s2g_ref[rows, :] = _dot(hg, w2)
        s2c_ref[rows, :] = _dot(hc, w2)

    @pl.when(p == 1)
    def _layer2():
        b2 = b2_ref[...]
        ones = jnp.ones((16, 16), jnp.float32)
        zg = _dot(adj_ref[...], s2g_ref[...]) + b2
        zc = _dot(adjc_ref[...], s2c_ref[...]) + b2
        sg = _dot(jnp.exp(zg), ones)
        sc = _dot(jnp.exp(zc), ones)
        zg_ref[sub, :] = zg
        lg_ref[sub, :] = zg - jnp.log(sg)
        zc_ref[sub, :] = zc
        lc_ref[sub, :] = zc - jnp.log(sc)


def kernel(x, adj, adj_CNN, W1, b1, W2, b2):
    nfeat = x.shape[1]
    nhid = W1.shape[1]
    ncls = W2.shape[1]
    b1r = b1.reshape(1, nhid)
    b2r = b2.reshape(1, ncls)

    grid = (2, R)
    blk_adj = pl.BlockSpec((BM, N), lambda p, i: (i, 0))
    const = lambda r, c: pl.BlockSpec((r, c), lambda p, i: (0, 0))
    blk_p0 = lambda c: pl.BlockSpec(
        (G * BM, c), lambda p, i: ((1 - p) * (i // G) + p * (RG - 1), 0))
    blk_p1 = lambda c: pl.BlockSpec((G * BM, c), lambda p, i: (p * (i // G), 0))

    out = pl.pallas_call(
        _gcn_kernel,
        grid=grid,
        in_specs=[const(N, nfeat), blk_adj, blk_adj,
                  const(nfeat, nhid), const(nhid, ncls),
                  const(1, nhid), const(1, ncls)],
        out_specs=[blk_p1(ncls), blk_p1(ncls), blk_p1(ncls), blk_p1(ncls),
                   blk_p0(nhid), blk_p0(nhid)],
        out_shape=[jax.ShapeDtypeStruct((N, ncls), jnp.float32)] * 4
                  + [jax.ShapeDtypeStruct((N, nhid), jnp.float32)] * 2,
        scratch_shapes=[
            pltpu.VMEM((N, nhid), jnp.float32),
            pltpu.VMEM((N, ncls), jnp.float32),
            pltpu.VMEM((N, ncls), jnp.float32),
        ],
        compiler_params=pltpu.CompilerParams(
            dimension_semantics=("arbitrary", "arbitrary"),
        ),
    )(x, adj, adj_CNN, W1, W2, b1r, b2r)

    lsm_g, z_g, lsm_c, z_c, h_g, h_c = out
    return (lsm_g, z_g, lsm_c, z_c, h_g, h_c)
